# baseline (speedup 1.0000x reference)
import functools

import jax
import jax.numpy as jnp
from jax import lax
from jax.experimental import pallas as pl
from jax.experimental.pallas import tpu as pltpu
from jax.experimental.pallas import tpu_sc as plsc

E = 6_400_000
N = 100_000
NW = 32
EPW = E // NW
NTAB = 100_096

C1 = 4_000
NCH1 = EPW // C1
G1 = C1 // 16
U1 = 5
UE1 = 10

C2 = 4_000
NCH2 = EPW // C2
G2 = C2 // 16
U2 = 5
UE2 = 25

_mesh = plsc.VectorSubcoreMesh(core_axis_name="c", subcore_axis_name="s")
_params = pltpu.CompilerParams(needs_layout_passes=False)


def _worker():
    return lax.axis_index("c") * 16 + lax.axis_index("s")


@functools.partial(
    pl.kernel,
    out_type=[jax.ShapeDtypeStruct((NW * NTAB,), jnp.float32),
              jax.ShapeDtypeStruct((E,), jnp.float32)],
    mesh=_mesh,
    scratch_types=[
        pltpu.VMEM((NTAB,), jnp.float32),
        pltpu.VMEM((C1,), jnp.int32),
        pltpu.VMEM((C1,), jnp.int32),
        pltpu.VMEM((C1,), jnp.float32),
        pltpu.VMEM((C1,), jnp.float32),
        pltpu.VMEM((C1,), jnp.float32),
        pltpu.VMEM((C1,), jnp.float32),
        pltpu.SemaphoreType.DMA((2,)),
        pltpu.SemaphoreType.DMA((2,)),
        pltpu.SemaphoreType.DMA((2,)),
    ],
    compiler_params=_params,
)
def _segment_sum(seg, w, out, ew_out, denom_v, idx_v0, idx_v1, w_v0, w_v1,
                 ew_v0, ew_v1, isem, wsem, esem):
    wid = _worker()
    idx_b = (idx_v0, idx_v1)
    w_b = (w_v0, w_v1)
    ew_b = (ew_v0, ew_v1)

    def start(j, b):
        base = wid * EPW + j * C1
        pltpu.async_copy(seg.at[pl.ds(base, C1)], idx_b[b], isem.at[b])
        pltpu.async_copy(w.at[pl.ds(base, C1)], w_b[b], wsem.at[b])

    start(0, 0)
    start(1, 1)

    zero16 = jnp.zeros((16,), jnp.float32)

    def zb(r, carry):
        for u in range(8):
            denom_v[pl.ds(r * 128 + u * 16, 16)] = zero16
        return carry
    lax.fori_loop(0, NTAB // 128, zb, 0)

    def pair(p, carry):
        for b in range(2):
            j = p * 2 + b
            pltpu.make_async_copy(seg.at[pl.ds(0, C1)], idx_b[b],
                                  isem.at[b]).wait()
            pltpu.make_async_copy(w.at[pl.ds(0, C1)], w_b[b],
                                  wsem.at[b]).wait()

            @pl.when(j >= 2)
            def _():
                pltpu.make_async_copy(ew_b[b], ew_out.at[pl.ds(0, C1)],
                                      esem.at[b]).wait()

            def expb(g, cc):
                for u in range(UE1):
                    o = (g * UE1 + u) * 16
                    ew_b[b][pl.ds(o, 16)] = jnp.exp(w_b[b][pl.ds(o, 16)])
                return cc
            lax.fori_loop(0, G1 // UE1, expb, 0)

            def grp(g, cc):
                for u in range(U1):
                    o = (g * U1 + u) * 16
                    idx = idx_b[b][pl.ds(o, 16)]
                    ew = ew_b[b][pl.ds(o, 16)]
                    plsc.addupdate_scatter(denom_v, [idx], ew)
                return cc
            lax.fori_loop(0, G1 // U1, grp, 0)

            @pl.when(j + 2 < NCH1)
            def _():
                start(j + 2, b)

            base = wid * EPW + j * C1
            pltpu.async_copy(ew_b[b], ew_out.at[pl.ds(base, C1)], esem.at[b])
        return carry
    lax.fori_loop(0, NCH1 // 2, pair, 0)

    for b in range(2):
        pltpu.make_async_copy(ew_b[b], ew_out.at[pl.ds(0, C1)],
                              esem.at[b]).wait()

    pltpu.sync_copy(denom_v, out.at[pl.ds(wid * NTAB, NTAB)])


@functools.partial(
    pl.kernel,
    out_type=jax.ShapeDtypeStruct((E,), jnp.float32),
    mesh=_mesh,
    scratch_types=[
        pltpu.VMEM((NTAB,), jnp.float32),
        pltpu.VMEM((C2,), jnp.int32),
        pltpu.VMEM((C2,), jnp.int32),
        pltpu.VMEM((C2,), jnp.float32),
        pltpu.VMEM((C2,), jnp.float32),
        pltpu.VMEM((C2,), jnp.float32),
        pltpu.VMEM((C2,), jnp.float32),
        pltpu.SemaphoreType.DMA((2,)),
        pltpu.SemaphoreType.DMA((2,)),
        pltpu.SemaphoreType.DMA((2,)),
        pltpu.SemaphoreType.DMA,
    ],
    compiler_params=_params,
)
def _normalize(seg, ew, denom_inv, out, denom_v, idx_v0, idx_v1, w_v0, w_v1,
               out_v0, out_v1, isem, wsem, osem, dsem):
    wid = _worker()
    idx_b = (idx_v0, idx_v1)
    w_b = (w_v0, w_v1)
    out_b = (out_v0, out_v1)

    ddenom = pltpu.async_copy(denom_inv, denom_v, dsem)

    def start(j, b):
        base = wid * EPW + j * C2
        pltpu.async_copy(seg.at[pl.ds(base, C2)], idx_b[b], isem.at[b])
        pltpu.async_copy(ew.at[pl.ds(base, C2)], w_b[b], wsem.at[b])

    start(0, 0)
    start(1, 1)
    ddenom.wait()

    def pair(p, carry):
        for b in range(2):
            j = p * 2 + b
            pltpu.make_async_copy(seg.at[pl.ds(0, C2)], idx_b[b],
                                  isem.at[b]).wait()
            pltpu.make_async_copy(ew.at[pl.ds(0, C2)], w_b[b],
                                  wsem.at[b]).wait()

            @pl.when(j >= 2)
            def _():
                pltpu.make_async_copy(out_b[b], out.at[pl.ds(0, C2)],
                                      osem.at[b]).wait()

            def grp(g, cc):
                for u in range(U2):
                    o = (g * U2 + u) * 16
                    idx = idx_b[b][pl.ds(o, 16)]
                    ew = w_b[b][pl.ds(o, 16)]
                    dinv = plsc.load_gather(denom_v, [idx])
                    out_b[b][pl.ds(o, 16)] = ew * dinv
                return cc
            lax.fori_loop(0, G2 // U2, grp, 0)

            base = wid * EPW + j * C2
            pltpu.async_copy(out_b[b], out.at[pl.ds(base, C2)], osem.at[b])

            @pl.when(j + 2 < NCH2)
            def _():
                start(j + 2, b)
        return carry
    lax.fori_loop(0, NCH2 // 2, pair, 0)

    for b in range(2):
        pltpu.make_async_copy(out_b[b], out.at[pl.ds(0, C2)],
                              osem.at[b]).wait()


def kernel(edge_index, edge_weight):
    seg = edge_index[1]
    partial, ew = _segment_sum(seg, edge_weight)
    acc = partial[0:NTAB]
    for t in range(1, NW):
        acc = acc + partial[t * NTAB:(t + 1) * NTAB]
    denom_inv = 1.0 / acc
    wnorm = _normalize(seg, ew, denom_inv)
    return (edge_index, wnorm)


# speedup vs baseline: 1.2440x; 1.2440x over previous
import functools

import jax
import jax.numpy as jnp
from jax import lax
from jax.experimental import pallas as pl
from jax.experimental.pallas import tpu as pltpu
from jax.experimental.pallas import tpu_sc as plsc

E = 6_400_000
N = 100_000
NW = 32
EPW = E // NW
NTAB = 100_096

C1 = 4_000
NCH1 = EPW // C1
G1 = C1 // 16
U1 = 10
UE1 = 10

C2 = 4_000
NCH2 = EPW // C2
G2 = C2 // 16
U2 = 10
UE2 = 25

_mesh = plsc.VectorSubcoreMesh(core_axis_name="c", subcore_axis_name="s")
_params = pltpu.CompilerParams(needs_layout_passes=False)


def _worker():
    return lax.axis_index("c") * 16 + lax.axis_index("s")


@functools.partial(
    pl.kernel,
    out_type=[jax.ShapeDtypeStruct((NW * NTAB,), jnp.float32),
              jax.ShapeDtypeStruct((E,), jnp.float32)],
    mesh=_mesh,
    scratch_types=[
        pltpu.VMEM((NTAB,), jnp.float32),
        pltpu.VMEM((C1,), jnp.int32),
        pltpu.VMEM((C1,), jnp.int32),
        pltpu.VMEM((C1,), jnp.float32),
        pltpu.VMEM((C1,), jnp.float32),
        pltpu.VMEM((C1,), jnp.float32),
        pltpu.VMEM((C1,), jnp.float32),
        pltpu.SemaphoreType.DMA((2,)),
        pltpu.SemaphoreType.DMA((2,)),
        pltpu.SemaphoreType.DMA((2,)),
    ],
    compiler_params=_params,
)
def _segment_sum(seg, w, out, ew_out, denom_v, idx_v0, idx_v1, w_v0, w_v1,
                 ew_v0, ew_v1, isem, wsem, esem):
    wid = _worker()
    idx_b = (idx_v0, idx_v1)
    w_b = (w_v0, w_v1)
    ew_b = (ew_v0, ew_v1)

    def start(j, b):
        base = wid * EPW + j * C1
        pltpu.async_copy(seg.at[pl.ds(base, C1)], idx_b[b], isem.at[b])
        pltpu.async_copy(w.at[pl.ds(base, C1)], w_b[b], wsem.at[b])

    start(0, 0)
    start(1, 1)

    zero16 = jnp.zeros((16,), jnp.float32)

    def zb(r, carry):
        for u in range(8):
            denom_v[pl.ds(r * 128 + u * 16, 16)] = zero16
        return carry
    lax.fori_loop(0, NTAB // 128, zb, 0)

    def pair(p, carry):
        for b in range(2):
            j = p * 2 + b
            pltpu.make_async_copy(seg.at[pl.ds(0, C1)], idx_b[b],
                                  isem.at[b]).wait()
            pltpu.make_async_copy(w.at[pl.ds(0, C1)], w_b[b],
                                  wsem.at[b]).wait()

            @pl.when(j >= 2)
            def _():
                pltpu.make_async_copy(ew_b[b], ew_out.at[pl.ds(0, C1)],
                                      esem.at[b]).wait()

            def expb(g, cc):
                for u in range(UE1):
                    o = (g * UE1 + u) * 16
                    ew_b[b][pl.ds(o, 16)] = jnp.exp(w_b[b][pl.ds(o, 16)])
                return cc
            lax.fori_loop(0, G1 // UE1, expb, 0)

            def grp(g, cc):
                o0 = g * U1 * 16
                idxs = [idx_b[b][pl.ds(o0 + u * 16, 16)]
                        for u in range(U1)]
                ews = [ew_b[b][pl.ds(o0 + u * 16, 16)]
                       for u in range(U1)]
                for u in range(U1):
                    plsc.addupdate_scatter(denom_v, [idxs[u]], ews[u])
                return cc
            lax.fori_loop(0, G1 // U1, grp, 0)

            @pl.when(j + 2 < NCH1)
            def _():
                start(j + 2, b)

            base = wid * EPW + j * C1
            pltpu.async_copy(ew_b[b], ew_out.at[pl.ds(base, C1)], esem.at[b])
        return carry
    lax.fori_loop(0, NCH1 // 2, pair, 0)

    for b in range(2):
        pltpu.make_async_copy(ew_b[b], ew_out.at[pl.ds(0, C1)],
                              esem.at[b]).wait()

    pltpu.sync_copy(denom_v, out.at[pl.ds(wid * NTAB, NTAB)])


@functools.partial(
    pl.kernel,
    out_type=jax.ShapeDtypeStruct((E,), jnp.float32),
    mesh=_mesh,
    scratch_types=[
        pltpu.VMEM((NTAB,), jnp.float32),
        pltpu.VMEM((C2,), jnp.int32),
        pltpu.VMEM((C2,), jnp.int32),
        pltpu.VMEM((C2,), jnp.float32),
        pltpu.VMEM((C2,), jnp.float32),
        pltpu.VMEM((C2,), jnp.float32),
        pltpu.VMEM((C2,), jnp.float32),
        pltpu.SemaphoreType.DMA((2,)),
        pltpu.SemaphoreType.DMA((2,)),
        pltpu.SemaphoreType.DMA((2,)),
        pltpu.SemaphoreType.DMA,
    ],
    compiler_params=_params,
)
def _normalize(seg, ew, denom_inv, out, denom_v, idx_v0, idx_v1, w_v0, w_v1,
               out_v0, out_v1, isem, wsem, osem, dsem):
    wid = _worker()
    idx_b = (idx_v0, idx_v1)
    w_b = (w_v0, w_v1)
    out_b = (out_v0, out_v1)

    ddenom = pltpu.async_copy(denom_inv, denom_v, dsem)

    def start(j, b):
        base = wid * EPW + j * C2
        pltpu.async_copy(seg.at[pl.ds(base, C2)], idx_b[b], isem.at[b])
        pltpu.async_copy(ew.at[pl.ds(base, C2)], w_b[b], wsem.at[b])

    start(0, 0)
    start(1, 1)
    ddenom.wait()

    def pair(p, carry):
        for b in range(2):
            j = p * 2 + b
            pltpu.make_async_copy(seg.at[pl.ds(0, C2)], idx_b[b],
                                  isem.at[b]).wait()
            pltpu.make_async_copy(ew.at[pl.ds(0, C2)], w_b[b],
                                  wsem.at[b]).wait()

            @pl.when(j >= 2)
            def _():
                pltpu.make_async_copy(out_b[b], out.at[pl.ds(0, C2)],
                                      osem.at[b]).wait()

            def grp(g, cc):
                o0 = g * U2 * 16
                idxs = [idx_b[b][pl.ds(o0 + u * 16, 16)]
                        for u in range(U2)]
                ews = [w_b[b][pl.ds(o0 + u * 16, 16)]
                       for u in range(U2)]
                dinvs = [plsc.load_gather(denom_v, [idxs[u]])
                         for u in range(U2)]
                for u in range(U2):
                    out_b[b][pl.ds(o0 + u * 16, 16)] = ews[u] * dinvs[u]
                return cc
            lax.fori_loop(0, G2 // U2, grp, 0)

            base = wid * EPW + j * C2
            pltpu.async_copy(out_b[b], out.at[pl.ds(base, C2)], osem.at[b])

            @pl.when(j + 2 < NCH2)
            def _():
                start(j + 2, b)
        return carry
    lax.fori_loop(0, NCH2 // 2, pair, 0)

    for b in range(2):
        pltpu.make_async_copy(out_b[b], out.at[pl.ds(0, C2)],
                              osem.at[b]).wait()


def kernel(edge_index, edge_weight):
    seg = edge_index[1]
    partial, ew = _segment_sum(seg, edge_weight)
    acc = partial[0:NTAB]
    for t in range(1, NW):
        acc = acc + partial[t * NTAB:(t + 1) * NTAB]
    denom_inv = 1.0 / acc
    wnorm = _normalize(seg, ew, denom_inv)
    return (edge_index, wnorm)


# speedup vs baseline: 1.9633x; 1.5782x over previous
import functools

import jax
import jax.numpy as jnp
from jax import lax
from jax.experimental import pallas as pl
from jax.experimental.pallas import tpu as pltpu
from jax.experimental.pallas import tpu_sc as plsc

E = 6_400_000
N = 100_000
NW = 32
EPW = E // NW
NTAB = 100_096

C1 = 4_000
NCH1 = EPW // C1
G1 = C1 // 16
U1 = 10
UE1 = 10

C2 = 4_000
NCH2 = EPW // C2
G2 = C2 // 16
U2 = 10
UE2 = 25

_mesh = plsc.VectorSubcoreMesh(core_axis_name="c", subcore_axis_name="s")
_params = pltpu.CompilerParams(needs_layout_passes=False)


def _worker():
    return lax.axis_index("c") * 16 + lax.axis_index("s")


@functools.partial(
    pl.kernel,
    out_type=[jax.ShapeDtypeStruct((NW * NTAB,), jnp.float32),
              jax.ShapeDtypeStruct((E,), jnp.float32)],
    mesh=_mesh,
    scratch_types=[
        pltpu.VMEM((NTAB,), jnp.float32),
        pltpu.VMEM((C1,), jnp.int32),
        pltpu.VMEM((C1,), jnp.int32),
        pltpu.VMEM((C1,), jnp.float32),
        pltpu.VMEM((C1,), jnp.float32),
        pltpu.VMEM((C1,), jnp.float32),
        pltpu.VMEM((C1,), jnp.float32),
        pltpu.SemaphoreType.DMA((2,)),
        pltpu.SemaphoreType.DMA((2,)),
        pltpu.SemaphoreType.DMA((2,)),
    ],
    compiler_params=_params,
)
def _segment_sum(seg, w, out, ew_out, denom_v, idx_v0, idx_v1, w_v0, w_v1,
                 ew_v0, ew_v1, isem, wsem, esem):
    wid = _worker()
    idx_b = (idx_v0, idx_v1)
    w_b = (w_v0, w_v1)
    ew_b = (ew_v0, ew_v1)

    def start(j, b):
        base = wid * EPW + j * C1
        pltpu.async_copy(seg.at[pl.ds(base, C1)], idx_b[b], isem.at[b])
        pltpu.async_copy(w.at[pl.ds(base, C1)], w_b[b], wsem.at[b])

    start(0, 0)
    start(1, 1)

    zero16 = jnp.zeros((16,), jnp.float32)

    def zb(r, carry):
        for u in range(8):
            denom_v[pl.ds(r * 128 + u * 16, 16)] = zero16
        return carry
    lax.fori_loop(0, NTAB // 128, zb, 0)

    def pair(p, carry):
        for b in range(2):
            j = p * 2 + b
            pltpu.make_async_copy(seg.at[pl.ds(0, C1)], idx_b[b],
                                  isem.at[b]).wait()
            pltpu.make_async_copy(w.at[pl.ds(0, C1)], w_b[b],
                                  wsem.at[b]).wait()

            @pl.when(j >= 2)
            def _():
                pltpu.make_async_copy(ew_b[b], ew_out.at[pl.ds(0, C1)],
                                      esem.at[b]).wait()

            def grp(g, cc):
                o0 = g * U1 * 16
                idxs = [idx_b[b][pl.ds(o0 + u * 16, 16)]
                        for u in range(U1)]
                ews = [jnp.exp(w_b[b][pl.ds(o0 + u * 16, 16)])
                       for u in range(U1)]
                for u in range(U1):
                    ew_b[b][pl.ds(o0 + u * 16, 16)] = ews[u]
                for u in range(U1):
                    plsc.addupdate_scatter(denom_v, [idxs[u]], ews[u])
                return cc
            lax.fori_loop(0, G1 // U1, grp, 0)

            @pl.when(j + 2 < NCH1)
            def _():
                start(j + 2, b)

            base = wid * EPW + j * C1
            pltpu.async_copy(ew_b[b], ew_out.at[pl.ds(base, C1)], esem.at[b])
        return carry
    lax.fori_loop(0, NCH1 // 2, pair, 0)

    for b in range(2):
        pltpu.make_async_copy(ew_b[b], ew_out.at[pl.ds(0, C1)],
                              esem.at[b]).wait()

    pltpu.sync_copy(denom_v, out.at[pl.ds(wid * NTAB, NTAB)])


@functools.partial(
    pl.kernel,
    out_type=jax.ShapeDtypeStruct((E,), jnp.float32),
    mesh=_mesh,
    scratch_types=[
        pltpu.VMEM((NTAB,), jnp.float32),
        pltpu.VMEM((C2,), jnp.int32),
        pltpu.VMEM((C2,), jnp.int32),
        pltpu.VMEM((C2,), jnp.float32),
        pltpu.VMEM((C2,), jnp.float32),
        pltpu.VMEM((C2,), jnp.float32),
        pltpu.VMEM((C2,), jnp.float32),
        pltpu.SemaphoreType.DMA((2,)),
        pltpu.SemaphoreType.DMA((2,)),
        pltpu.SemaphoreType.DMA((2,)),
        pltpu.SemaphoreType.DMA,
    ],
    compiler_params=_params,
)
def _normalize(seg, ew, denom_inv, out, denom_v, idx_v0, idx_v1, w_v0, w_v1,
               out_v0, out_v1, isem, wsem, osem, dsem):
    wid = _worker()
    idx_b = (idx_v0, idx_v1)
    w_b = (w_v0, w_v1)
    out_b = (out_v0, out_v1)

    ddenom = pltpu.async_copy(denom_inv, denom_v, dsem)

    def start(j, b):
        base = wid * EPW + j * C2
        pltpu.async_copy(seg.at[pl.ds(base, C2)], idx_b[b], isem.at[b])
        pltpu.async_copy(ew.at[pl.ds(base, C2)], w_b[b], wsem.at[b])

    start(0, 0)
    start(1, 1)
    ddenom.wait()

    def pair(p, carry):
        for b in range(2):
            j = p * 2 + b
            pltpu.make_async_copy(seg.at[pl.ds(0, C2)], idx_b[b],
                                  isem.at[b]).wait()
            pltpu.make_async_copy(ew.at[pl.ds(0, C2)], w_b[b],
                                  wsem.at[b]).wait()

            @pl.when(j >= 2)
            def _():
                pltpu.make_async_copy(out_b[b], out.at[pl.ds(0, C2)],
                                      osem.at[b]).wait()

            def grp(g, cc):
                o0 = g * U2 * 16
                idxs = [idx_b[b][pl.ds(o0 + u * 16, 16)]
                        for u in range(U2)]
                ews = [w_b[b][pl.ds(o0 + u * 16, 16)]
                       for u in range(U2)]
                dinvs = [plsc.load_gather(denom_v, [idxs[u]])
                         for u in range(U2)]
                for u in range(U2):
                    out_b[b][pl.ds(o0 + u * 16, 16)] = ews[u] * dinvs[u]
                return cc
            lax.fori_loop(0, G2 // U2, grp, 0)

            base = wid * EPW + j * C2
            pltpu.async_copy(out_b[b], out.at[pl.ds(base, C2)], osem.at[b])

            @pl.when(j + 2 < NCH2)
            def _():
                start(j + 2, b)
        return carry
    lax.fori_loop(0, NCH2 // 2, pair, 0)

    for b in range(2):
        pltpu.make_async_copy(out_b[b], out.at[pl.ds(0, C2)],
                              osem.at[b]).wait()


def kernel(edge_index, edge_weight):
    seg = edge_index[1]
    partial, ew = _segment_sum(seg, edge_weight)
    acc = partial[0:NTAB]
    for t in range(1, NW):
        acc = acc + partial[t * NTAB:(t + 1) * NTAB]
    denom_inv = 1.0 / acc
    wnorm = _normalize(seg, ew, denom_inv)
    return (edge_index, wnorm)
